# baseline (speedup 1.0000x reference)
import functools

import jax
import jax.numpy as jnp
from jax import lax
from jax.experimental import pallas as pl
from jax.experimental.pallas import tpu as pltpu
from jax.experimental.pallas import tpu_sc as plsc

N = 100000
D = 128
S = 2048

CHUNK = 160
NCHUNKS = N // CHUNK
NW = 32
BASE = NCHUNKS // NW
EXTRA = NCHUNKS - BASE * NW


def _merge_body(num_ref, den_ref, o_ref):
    num = num_ref[0] + num_ref[1]
    den = jnp.sum(den_ref[...], axis=0)
    o_ref[...] = num / (den[:, None] + 1e-16)


def _pool_body(x_hbm, b_hbm, a_hbm, z_hbm, num_hbm, den_hbm,
               x_buf0, b_buf0, x_buf1, b_buf1,
               sem0, sem1, a_buf, stage, idx1, den_loc, acc_sh):
    cid = lax.axis_index("c")
    sid = lax.axis_index("s")
    wid = sid * 2 + cid

    z16 = jnp.zeros((16,), jnp.float32)

    pltpu.sync_copy(a_hbm, a_buf)
    pltpu.sync_copy(z_hbm.at[pl.ds(sid * 128, 128)],
                    acc_sh.at[pl.ds(sid * 128, 128)])

    def zden(i, _):
        den_loc[pl.ds(i * 16, 16)] = z16
        return 0
    lax.fori_loop(0, S // 16, zden, 0)
    plsc.subcore_barrier()

    iota16 = jnp.arange(16, dtype=jnp.int32)
    lane0 = iota16 == 0
    zi16 = jnp.zeros((16,), jnp.int32)

    def exp16(sv):
        y = sv * jnp.float32(1.4426950408889634)
        big = jnp.float32(12582912.0)
        r = (y + big) - big
        t = (y - r) * jnp.float32(0.6931471805599453)
        p = jnp.float32(1 / 720)
        for c in (1 / 120, 1 / 24, 1 / 6, 0.5, 1.0, 1.0):
            p = p * t + jnp.float32(c)
        n = r.astype(jnp.int32)
        scale = plsc.bitcast((n + 127) << 23, jnp.float32)
        return p * scale

    def flush(cur_seg, den_run, accs):
        for k in range(8):
            plsc.store_scatter(stage, [zi16, k * 16 + iota16], accs[k])
        plsc.store_scatter(idx1, [zi16],
                           jnp.full((16,), cur_seg, jnp.int32), mask=lane0)
        pltpu.sync_copy(stage, acc_sh.at[idx1], add=True)
        plsc.addupdate_scatter(den_loc, [jnp.full((16,), cur_seg, jnp.int32)],
                               den_run, mask=lane0)

    def start_dma(c, x_buf, b_buf, sem):
        pltpu.async_copy(x_hbm.at[pl.ds(c * CHUNK * D, CHUNK * D)], x_buf, sem)
        pltpu.async_copy(b_hbm.at[pl.ds(c * CHUNK, CHUNK)], b_buf, sem)

    def wait_dma(x_buf, b_buf, sem):
        pltpu.make_async_copy(x_hbm.at[pl.ds(0, CHUNK * D)], x_buf, sem).wait()
        pltpu.make_async_copy(b_hbm.at[pl.ds(0, CHUNK)], b_buf, sem).wait()

    def rnd(u):
        ui = plsc.bitcast(u, jnp.int32)
        lsb = (ui >> 16) & 1
        r = (ui + jnp.int32(0x7FFF) + lsb) & jnp.int32(-65536)
        return plsc.bitcast(r, jnp.float32)

    def rnd8(vs):
        return [rnd(v) for v in vs]

    def make_process(x_buf, b_buf, sem):
        def group_fn(g, carry):
            bvec = b_buf[pl.ds(g * 16, 16)]
            avs = rnd8([a_buf[pl.ds(k * 16, 16)] for k in range(8)])
            for j in range(16):
                cur_seg = carry[0]
                den_run = carry[1]
                accs = carry[2:]
                seg = bvec[j]
                changed = seg != cur_seg

                @pl.when(jnp.logical_and(changed, cur_seg >= 0))
                def _():
                    flush(cur_seg, den_run, accs)

                rbase = (g * 16 + j) * D
                xvs = [x_buf[pl.ds(rbase + k * 16, 16)] for k in range(8)]
                xrs = rnd8(xvs)
                pr = xrs[0] * avs[0]
                for k in range(1, 8):
                    pr = pr + xrs[k] * avs[k]
                t = jnp.sum(pr)
                s = jnp.where(t >= 0, t, jnp.float32(0.2) * t)
                evv = exp16(jnp.full((16,), s))

                m16 = jnp.full((16,), changed)
                new_accs = tuple(
                    jnp.where(m16, z16, a) + evv * xvs[k]
                    for k, a in enumerate(accs))
                den_run = jnp.where(m16, z16, den_run) + evv
                carry = (seg, den_run) + new_accs
            return carry

        def process(carry):
            wait_dma(x_buf, b_buf, sem)
            return lax.fori_loop(0, CHUNK // 16, group_fn, carry)
        return process

    proc0 = make_process(x_buf0, b_buf0, sem0)
    proc1 = make_process(x_buf1, b_buf1, sem1)

    nchunks_w = jnp.where(wid < EXTRA, BASE + 1, BASE)

    def chunk_fn(t, carry):
        nxt = wid + (t + 1) * NW

        @pl.when(jnp.logical_and(t + 1 < nchunks_w, (t + 1) % 2 == 0))
        def _():
            start_dma(nxt, x_buf0, b_buf0, sem0)

        @pl.when(jnp.logical_and(t + 1 < nchunks_w, (t + 1) % 2 == 1))
        def _():
            start_dma(nxt, x_buf1, b_buf1, sem1)

        return lax.cond(t % 2 == 0, proc0, proc1, carry)

    carry0 = (jnp.int32(-1), z16) + (z16,) * 8
    start_dma(wid, x_buf0, b_buf0, sem0)
    carry = lax.fori_loop(0, nchunks_w, chunk_fn, carry0)

    @pl.when(carry[0] >= 0)
    def _():
        flush(carry[0], carry[1], carry[2:])

    plsc.subcore_barrier()
    pltpu.sync_copy(acc_sh.at[pl.ds(sid * 128, 128)],
                    num_hbm.at[cid, pl.ds(sid * 128, 128)])
    pltpu.sync_copy(den_loc, den_hbm.at[pl.ds(wid * S, S)])


_pool = functools.partial(
    pl.kernel,
    out_type=(jax.ShapeDtypeStruct((2, S, D), jnp.float32),
              jax.ShapeDtypeStruct((NW * S,), jnp.float32)),
    mesh=plsc.VectorSubcoreMesh(core_axis_name="c", subcore_axis_name="s"),
    compiler_params=pltpu.CompilerParams(needs_layout_passes=False),
    scratch_types=[
        pltpu.VMEM((CHUNK * D,), jnp.float32),
        pltpu.VMEM((CHUNK,), jnp.int32),
        pltpu.VMEM((CHUNK * D,), jnp.float32),
        pltpu.VMEM((CHUNK,), jnp.int32),
        pltpu.SemaphoreType.DMA,
        pltpu.SemaphoreType.DMA,
        pltpu.VMEM((D,), jnp.float32),
        pltpu.VMEM((1, D), jnp.float32),
        pltpu.VMEM((1,), jnp.int32),
        pltpu.VMEM((S,), jnp.float32),
        pltpu.VMEM_SHARED((S, D), jnp.float32),
    ],
)(_pool_body)


def kernel(x, batch, attention_vector):
    zeros2d = jnp.zeros((S, D), jnp.float32)
    num_parts, den_flat = _pool(x.reshape(N * D), batch.astype(jnp.int32),
                                attention_vector, zeros2d)
    den_parts = den_flat.reshape(NW, S)
    return pl.pallas_call(
        _merge_body,
        out_shape=jax.ShapeDtypeStruct((S, D), jnp.float32),
    )(num_parts, den_parts)


# speedup vs baseline: 1.3575x; 1.3575x over previous
import functools

import jax
import jax.numpy as jnp
from jax import lax
from jax.experimental import pallas as pl
from jax.experimental.pallas import tpu as pltpu
from jax.experimental.pallas import tpu_sc as plsc

N = 100000
D = 128
S = 2048

CHUNK = 160
NCHUNKS = N // CHUNK
NW = 32
BASE = NCHUNKS // NW
EXTRA = NCHUNKS - BASE * NW


def _merge_body(num_ref, den_ref, o_ref):
    num = num_ref[0] + num_ref[1]
    den = jnp.sum(den_ref[...], axis=0)
    o_ref[...] = num / (den[:, None] + 1e-16)


def _pool_body(x_hbm, b_hbm, a_hbm, z_hbm, num_hbm, den_hbm,
               x_buf0, b_buf0, x_buf1, b_buf1,
               sem0, sem1, a_buf, stage, idx1, den_loc, acc_sh):
    cid = lax.axis_index("c")
    sid = lax.axis_index("s")
    wid = sid * 2 + cid

    z16 = jnp.zeros((16,), jnp.float32)

    pltpu.sync_copy(a_hbm, a_buf)
    pltpu.sync_copy(z_hbm.at[pl.ds(sid * 128, 128)],
                    acc_sh.at[pl.ds(sid * 128, 128)])

    def zden(i, _):
        den_loc[pl.ds(i * 16, 16)] = z16
        return 0
    lax.fori_loop(0, S // 16, zden, 0)
    plsc.subcore_barrier()

    iota16 = jnp.arange(16, dtype=jnp.int32)
    lane0 = iota16 == 0
    zi16 = jnp.zeros((16,), jnp.int32)

    def exp16(sv):
        y = sv * jnp.float32(1.4426950408889634)
        big = jnp.float32(12582912.0)
        r = (y + big) - big
        t = (y - r) * jnp.float32(0.6931471805599453)
        p = jnp.float32(1 / 720)
        for c in (1 / 120, 1 / 24, 1 / 6, 0.5, 1.0, 1.0):
            p = p * t + jnp.float32(c)
        n = r.astype(jnp.int32)
        scale = plsc.bitcast((n + 127) << 23, jnp.float32)
        return p * scale

    def flush(cur_seg, den_run, accs):
        for k in range(8):
            plsc.store_scatter(stage, [zi16, k * 16 + iota16], accs[k])
        plsc.store_scatter(idx1, [zi16],
                           jnp.full((16,), cur_seg, jnp.int32), mask=lane0)
        pltpu.sync_copy(stage, acc_sh.at[idx1], add=True)
        plsc.addupdate_scatter(den_loc, [jnp.full((16,), cur_seg, jnp.int32)],
                               den_run, mask=lane0)

    def start_dma(c, x_buf, b_buf, sem):
        pltpu.async_copy(x_hbm.at[pl.ds(c * CHUNK * D, CHUNK * D)], x_buf, sem)
        pltpu.async_copy(b_hbm.at[pl.ds(c * CHUNK, CHUNK)], b_buf, sem)

    def wait_dma(x_buf, b_buf, sem):
        pltpu.make_async_copy(x_hbm.at[pl.ds(0, CHUNK * D)], x_buf, sem).wait()
        pltpu.make_async_copy(b_hbm.at[pl.ds(0, CHUNK)], b_buf, sem).wait()

    def rnd(u):
        ui = plsc.bitcast(u, jnp.int32)
        r = (ui + jnp.int32(0x8000)) & jnp.int32(-65536)
        return plsc.bitcast(r, jnp.float32)

    def rnd8(vs):
        return [rnd(v) for v in vs]

    def make_process(x_buf, b_buf, sem):
        def group_fn(g, carry):
            bvec = b_buf[pl.ds(g * 16, 16)]
            avs = rnd8([a_buf[pl.ds(k * 16, 16)] for k in range(8)])
            for j in range(16):
                cur_seg = carry[0]
                den_run = carry[1]
                accs = carry[2:]
                seg = bvec[j]
                changed = seg != cur_seg

                @pl.when(jnp.logical_and(changed, cur_seg >= 0))
                def _():
                    flush(cur_seg, den_run, accs)

                rbase = (g * 16 + j) * D
                xvs = [x_buf[pl.ds(rbase + k * 16, 16)] for k in range(8)]
                xrs = rnd8(xvs)
                pr = xrs[0] * avs[0]
                for k in range(1, 8):
                    pr = pr + xrs[k] * avs[k]
                t = jnp.sum(pr)
                s = jnp.where(t >= 0, t, jnp.float32(0.2) * t)
                evv = jnp.exp(jnp.full((16,), s))

                m16 = jnp.full((16,), changed)
                new_accs = tuple(
                    jnp.where(m16, z16, a) + evv * xvs[k]
                    for k, a in enumerate(accs))
                den_run = jnp.where(m16, z16, den_run) + evv
                carry = (seg, den_run) + new_accs
            return carry

        def process(carry):
            wait_dma(x_buf, b_buf, sem)
            return lax.fori_loop(0, CHUNK // 16, group_fn, carry)
        return process

    proc0 = make_process(x_buf0, b_buf0, sem0)
    proc1 = make_process(x_buf1, b_buf1, sem1)

    nchunks_w = jnp.where(wid < EXTRA, BASE + 1, BASE)

    def chunk_fn(t, carry):
        nxt = wid + (t + 1) * NW

        @pl.when(jnp.logical_and(t + 1 < nchunks_w, (t + 1) % 2 == 0))
        def _():
            start_dma(nxt, x_buf0, b_buf0, sem0)

        @pl.when(jnp.logical_and(t + 1 < nchunks_w, (t + 1) % 2 == 1))
        def _():
            start_dma(nxt, x_buf1, b_buf1, sem1)

        return lax.cond(t % 2 == 0, proc0, proc1, carry)

    carry0 = (jnp.int32(-1), z16) + (z16,) * 8
    start_dma(wid, x_buf0, b_buf0, sem0)
    carry = lax.fori_loop(0, nchunks_w, chunk_fn, carry0)

    @pl.when(carry[0] >= 0)
    def _():
        flush(carry[0], carry[1], carry[2:])

    plsc.subcore_barrier()
    pltpu.sync_copy(acc_sh.at[pl.ds(sid * 128, 128)],
                    num_hbm.at[cid, pl.ds(sid * 128, 128)])
    pltpu.sync_copy(den_loc, den_hbm.at[pl.ds(wid * S, S)])


_pool = functools.partial(
    pl.kernel,
    out_type=(jax.ShapeDtypeStruct((2, S, D), jnp.float32),
              jax.ShapeDtypeStruct((NW * S,), jnp.float32)),
    mesh=plsc.VectorSubcoreMesh(core_axis_name="c", subcore_axis_name="s"),
    compiler_params=pltpu.CompilerParams(needs_layout_passes=False),
    scratch_types=[
        pltpu.VMEM((CHUNK * D,), jnp.float32),
        pltpu.VMEM((CHUNK,), jnp.int32),
        pltpu.VMEM((CHUNK * D,), jnp.float32),
        pltpu.VMEM((CHUNK,), jnp.int32),
        pltpu.SemaphoreType.DMA,
        pltpu.SemaphoreType.DMA,
        pltpu.VMEM((D,), jnp.float32),
        pltpu.VMEM((1, D), jnp.float32),
        pltpu.VMEM((1,), jnp.int32),
        pltpu.VMEM((S,), jnp.float32),
        pltpu.VMEM_SHARED((S, D), jnp.float32),
    ],
)(_pool_body)


def kernel(x, batch, attention_vector):
    zeros2d = jnp.zeros((S, D), jnp.float32)
    num_parts, den_flat = _pool(x.reshape(N * D), batch.astype(jnp.int32),
                                attention_vector, zeros2d)
    den_parts = den_flat.reshape(NW, S)
    return pl.pallas_call(
        _merge_body,
        out_shape=jax.ShapeDtypeStruct((S, D), jnp.float32),
    )(num_parts, den_parts)


# speedup vs baseline: 1.5835x; 1.1665x over previous
import functools

import jax
import jax.numpy as jnp
from jax import lax
from jax.experimental import pallas as pl
from jax.experimental.pallas import tpu as pltpu
from jax.experimental.pallas import tpu_sc as plsc

N = 100000
D = 128
S = 2048

CHUNK = 160
NCHUNKS = N // CHUNK
NW = 32
BASE = NCHUNKS // NW
EXTRA = NCHUNKS - BASE * NW


def _merge_body(num_ref, den_ref, o_ref):
    num = num_ref[0] + num_ref[1]
    den = jnp.sum(den_ref[...], axis=0)
    o_ref[...] = num / (den[:, None] + 1e-16)


def _pool_body(x_hbm, b_hbm, a_hbm, z_hbm, num_hbm, den_hbm,
               x_buf0, b_buf0, x_buf1, b_buf1,
               sem0, sem1, a_buf, stage, idx1, den_loc, acc_sh):
    cid = lax.axis_index("c")
    sid = lax.axis_index("s")
    wid = sid * 2 + cid

    z16 = jnp.zeros((16,), jnp.float32)

    pltpu.sync_copy(a_hbm, a_buf)
    pltpu.sync_copy(z_hbm.at[pl.ds(sid * 128, 128)],
                    acc_sh.at[pl.ds(sid * 128, 128)])

    def zden(i, _):
        den_loc[pl.ds(i * 16, 16)] = z16
        return 0
    lax.fori_loop(0, S // 16, zden, 0)
    plsc.subcore_barrier()

    iota16 = jnp.arange(16, dtype=jnp.int32)
    lane0 = iota16 == 0
    zi16 = jnp.zeros((16,), jnp.int32)

    def flush(cur_seg, den_run, accs):
        for k in range(8):
            plsc.store_scatter(stage, [zi16, k * 16 + iota16], accs[k])
        plsc.store_scatter(idx1, [zi16],
                           jnp.full((16,), cur_seg, jnp.int32), mask=lane0)
        pltpu.sync_copy(stage, acc_sh.at[idx1], add=True)
        plsc.addupdate_scatter(den_loc, [jnp.full((16,), cur_seg, jnp.int32)],
                               den_run, mask=lane0)

    def start_dma(c, x_buf, b_buf, sem):
        pltpu.async_copy(x_hbm.at[pl.ds(c * CHUNK * D, CHUNK * D)], x_buf, sem)
        pltpu.async_copy(b_hbm.at[pl.ds(c * CHUNK, CHUNK)], b_buf, sem)

    def wait_dma(x_buf, b_buf, sem):
        pltpu.make_async_copy(x_hbm.at[pl.ds(0, CHUNK * D)], x_buf, sem).wait()
        pltpu.make_async_copy(b_hbm.at[pl.ds(0, CHUNK)], b_buf, sem).wait()

    def rnd(u):
        ui = plsc.bitcast(u, jnp.int32)
        r = (ui + jnp.int32(0x8000)) & jnp.int32(-65536)
        return plsc.bitcast(r, jnp.float32)

    def rnd8(vs):
        return [rnd(v) for v in vs]

    def score_ev(xvs, avs):
        xrs = rnd8(xvs)
        pr = xrs[0] * avs[0]
        for k in range(1, 8):
            pr = pr + xrs[k] * avs[k]
        t = jnp.sum(pr)
        s = jnp.where(t >= 0, t, jnp.float32(0.2) * t)
        return jnp.exp(jnp.full((16,), s))

    def make_process(x_buf, b_buf, sem):
        def group_fn(g, carry):
            bvec = b_buf[pl.ds(g * 16, 16)]
            avs = rnd8([a_buf[pl.ds(k * 16, 16)] for k in range(8)])

            def fast_fn(carry):
                cur_seg = carry[0]
                den_run = carry[1]
                accs = list(carry[2:])
                for j in range(16):
                    rbase = (g * 16 + j) * D
                    xvs = [x_buf[pl.ds(rbase + k * 16, 16)]
                           for k in range(8)]
                    evv = score_ev(xvs, avs)
                    accs = [a + evv * xvs[k] for k, a in enumerate(accs)]
                    den_run = den_run + evv
                return (cur_seg, den_run) + tuple(accs)

            def slow_fn(carry):
                for j in range(16):
                    cur_seg = carry[0]
                    den_run = carry[1]
                    accs = carry[2:]
                    seg = bvec[j]
                    changed = seg != cur_seg

                    @pl.when(jnp.logical_and(changed, cur_seg >= 0))
                    def _():
                        flush(cur_seg, den_run, accs)

                    rbase = (g * 16 + j) * D
                    xvs = [x_buf[pl.ds(rbase + k * 16, 16)]
                           for k in range(8)]
                    evv = score_ev(xvs, avs)
                    m16 = jnp.full((16,), changed)
                    new_accs = tuple(
                        jnp.where(m16, z16, a) + evv * xvs[k]
                        for k, a in enumerate(accs))
                    den_run = jnp.where(m16, z16, den_run) + evv
                    carry = (seg, den_run) + new_accs
                return carry

            fast = jnp.logical_and(bvec[0] == bvec[15], bvec[0] == carry[0])
            return lax.cond(fast, fast_fn, slow_fn, carry)

        def process(carry):
            wait_dma(x_buf, b_buf, sem)
            return lax.fori_loop(0, CHUNK // 16, group_fn, carry)
        return process

    proc0 = make_process(x_buf0, b_buf0, sem0)
    proc1 = make_process(x_buf1, b_buf1, sem1)

    nchunks_w = jnp.where(wid < EXTRA, BASE + 1, BASE)

    def chunk_fn(t, carry):
        nxt = wid + (t + 1) * NW

        @pl.when(jnp.logical_and(t + 1 < nchunks_w, (t + 1) % 2 == 0))
        def _():
            start_dma(nxt, x_buf0, b_buf0, sem0)

        @pl.when(jnp.logical_and(t + 1 < nchunks_w, (t + 1) % 2 == 1))
        def _():
            start_dma(nxt, x_buf1, b_buf1, sem1)

        return lax.cond(t % 2 == 0, proc0, proc1, carry)

    carry0 = (jnp.int32(-1), z16) + (z16,) * 8
    start_dma(wid, x_buf0, b_buf0, sem0)
    carry = lax.fori_loop(0, nchunks_w, chunk_fn, carry0)

    @pl.when(carry[0] >= 0)
    def _():
        flush(carry[0], carry[1], carry[2:])

    plsc.subcore_barrier()
    pltpu.sync_copy(acc_sh.at[pl.ds(sid * 128, 128)],
                    num_hbm.at[cid, pl.ds(sid * 128, 128)])
    pltpu.sync_copy(den_loc, den_hbm.at[pl.ds(wid * S, S)])


_pool = functools.partial(
    pl.kernel,
    out_type=(jax.ShapeDtypeStruct((2, S, D), jnp.float32),
              jax.ShapeDtypeStruct((NW * S,), jnp.float32)),
    mesh=plsc.VectorSubcoreMesh(core_axis_name="c", subcore_axis_name="s"),
    compiler_params=pltpu.CompilerParams(needs_layout_passes=False),
    scratch_types=[
        pltpu.VMEM((CHUNK * D,), jnp.float32),
        pltpu.VMEM((CHUNK,), jnp.int32),
        pltpu.VMEM((CHUNK * D,), jnp.float32),
        pltpu.VMEM((CHUNK,), jnp.int32),
        pltpu.SemaphoreType.DMA,
        pltpu.SemaphoreType.DMA,
        pltpu.VMEM((D,), jnp.float32),
        pltpu.VMEM((1, D), jnp.float32),
        pltpu.VMEM((1,), jnp.int32),
        pltpu.VMEM((S,), jnp.float32),
        pltpu.VMEM_SHARED((S, D), jnp.float32),
    ],
)(_pool_body)


def kernel(x, batch, attention_vector):
    zeros2d = jnp.zeros((S, D), jnp.float32)
    num_parts, den_flat = _pool(x.reshape(N * D), batch.astype(jnp.int32),
                                attention_vector, zeros2d)
    den_parts = den_flat.reshape(NW, S)
    return pl.pallas_call(
        _merge_body,
        out_shape=jax.ShapeDtypeStruct((S, D), jnp.float32),
    )(num_parts, den_parts)
